# Initial kernel scaffold; baseline (speedup 1.0000x reference)
import jax
import jax.numpy as jnp
from jax.experimental import pallas as pl


def kernel(logits, target):
    raise NotImplementedError("write your pallas kernel here")


# speedup vs baseline: 25.9806x; 25.9806x over previous
import functools

import jax
import jax.numpy as jnp
from jax import lax
from jax.experimental import pallas as pl
from jax.experimental.pallas import tpu as pltpu
from jax.experimental.pallas import tpu_sc as plsc

_B, _C, _H, _W = 8, 2, 512, 512
_N = _B * _H * _W
_BLK = _H * _W
_NW = 32
_PER_W = _N // _NW
_CH = 16384
_NCH = _PER_W // _CH
_L = 16

_Q = (0.9999970510848344, -0.4998254028857509, 0.33078744547883354,
      -0.234172411224585, 0.14810505362112691, -0.06576904117079967,
      0.014026606298625958)


def _bce(xv, tf):
    ax = jnp.abs(xv)
    e = jnp.exp(-ax)
    q = jnp.float32(_Q[6]) * e + jnp.float32(_Q[5])
    for c in _Q[4::-1]:
        q = q * e + jnp.float32(c)
    return jnp.maximum(xv, jnp.float32(0.0)) + e * q - tf * xv


def _sc_reduce_body(x_hbm, t_hbm, out_hbm, xb, tb, accb):
    wid = lax.axis_index("s") * 2 + lax.axis_index("c")
    tbase = wid * _PER_W
    xbase = tbase + (wid // (_BLK // _PER_W) + 1) * _BLK

    def chunk_body(c, carry):
        pltpu.sync_copy(x_hbm.at[pl.ds(xbase + c * _CH, _CH)], xb)
        pltpu.sync_copy(t_hbm.at[pl.ds(tbase + c * _CH, _CH)], tb)

        def vec_body(i, acc):
            s_all, s_pos, c_pos = acc
            xv = xb[pl.ds(i * _L, _L)]
            tf = tb[pl.ds(i * _L, _L)].astype(jnp.float32)
            bce = _bce(xv, tf)
            pb = tf * bce
            return (s_all + bce, s_pos + pb, c_pos + tf)

        return lax.fori_loop(0, _CH // _L, vec_body, carry)

    z = jnp.zeros((_L,), jnp.float32)
    s_all, s_pos, c_pos = lax.fori_loop(0, _NCH, chunk_body, (z, z, z))
    accb[pl.ds(0, _L)] = s_all
    accb[pl.ds(_L, _L)] = s_pos
    accb[pl.ds(2 * _L, _L)] = c_pos
    pltpu.sync_copy(accb, out_hbm.at[wid])


@functools.lru_cache(maxsize=None)
def _sc_reduce():
    return pl.kernel(
        _sc_reduce_body,
        out_type=jax.ShapeDtypeStruct((_NW, 48), jnp.float32),
        mesh=plsc.VectorSubcoreMesh(core_axis_name="c", subcore_axis_name="s"),
        scratch_types=[
            pltpu.VMEM((_CH,), jnp.float32),
            pltpu.VMEM((_CH,), jnp.int32),
            pltpu.VMEM((48,), jnp.float32),
        ],
    )


_ROWS, _COLS = 2048, 1024
_RB = 128
_GRID = _ROWS // _RB


def _topk_body(k_ref, x_ref, t_ref, out_ref, keys):
    gi = pl.program_id(0)
    xv = x_ref[...]
    tv = t_ref[...]
    tf = tv.astype(jnp.float32)
    bce = jnp.maximum(xv, 0.0) + jnp.log1p(jnp.exp(-jnp.abs(xv))) - tf * xv
    keys[pl.ds(gi * _RB, _RB), :] = jnp.where(
        tv > 0, jnp.int32(0), pltpu.bitcast(bce, jnp.int32))

    @pl.when(gi == _GRID - 1)
    def _():
        k = k_ref[0]

        def bit_body(i, prefix):
            cand = prefix | jnp.left_shift(jnp.int32(1), 30 - i)
            cnt = jnp.sum((keys[...] >= cand).astype(jnp.int32))
            return jnp.where(cnt >= k, cand, prefix)

        prefix = lax.fori_loop(0, 31, bit_body, jnp.int32(0))
        kv = keys[...]
        vals = pltpu.bitcast(kv, jnp.float32)
        gt = kv > prefix
        cnt_gt = jnp.sum(gt.astype(jnp.int32))
        sum_gt = jnp.sum(jnp.where(gt, vals, 0.0))
        thr = jnp.max(jnp.where(kv == prefix, vals, 0.0))
        out_ref[0] = sum_gt + (k - cnt_gt).astype(jnp.float32) * thr


_topk_call = pl.pallas_call(
    _topk_body,
    grid=(_GRID,),
    in_specs=[
        pl.BlockSpec(memory_space=pltpu.SMEM),
        pl.BlockSpec((_RB, _COLS), lambda i: (i, 0)),
        pl.BlockSpec((_RB, _COLS), lambda i: (i, 0)),
    ],
    out_specs=pl.BlockSpec(memory_space=pltpu.SMEM),
    out_shape=jax.ShapeDtypeStruct((1,), jnp.float32),
    scratch_shapes=[pltpu.VMEM((_ROWS, _COLS), jnp.int32)],
)


def kernel(logits, target):
    xflat = logits.reshape(-1)
    tflat = target.reshape(-1)
    parts = _sc_reduce()(xflat, tflat)
    s_all = jnp.sum(parts[:, 0:16])
    s_pos = jnp.sum(parts[:, 16:32])
    n_pos_f = jnp.sum(parts[:, 32:48])

    num_pos = n_pos_f.astype(jnp.int32)
    num_neg = jnp.int32(_N) - num_pos
    k_pos = jnp.minimum(num_neg, 20 * num_pos)
    k_empty = jnp.maximum(
        1, (num_neg.astype(jnp.float32) * jnp.float32(0.01)).astype(jnp.int32))
    k = jnp.where(num_pos > 0, k_pos, k_empty)

    def fast(_):
        return s_all - s_pos

    def slow(_):
        x2d = logits[:, 1, :, :].reshape(_ROWS, _COLS)
        t2d = target.reshape(_ROWS, _COLS)
        return _topk_call(k.reshape(1), x2d, t2d)[0]

    topk_sum = lax.cond(k == num_neg, fast, slow, None)
    neg_keep = jnp.where(
        num_neg > 0, topk_sum / jnp.maximum(k, 1).astype(jnp.float32), 0.0)
    pos_keep = jnp.where(
        num_pos > 0, s_pos / jnp.maximum(n_pos_f, 1.0), 0.0)
    return pos_keep + neg_keep


# speedup vs baseline: 35.7612x; 1.3765x over previous
import functools

import jax
import jax.numpy as jnp
from jax import lax
from jax.experimental import pallas as pl
from jax.experimental.pallas import tpu as pltpu
from jax.experimental.pallas import tpu_sc as plsc

_B, _C, _H, _W = 8, 2, 512, 512
_N = _B * _H * _W
_NW = 32
_WPB = _NW // _B
_RPW = _H // _WPB
_CHR = 32
_NCH = _RPW // _CHR
_L = 16

_Q = (0.9999970510848344, -0.4998254028857509, 0.33078744547883354,
      -0.234172411224585, 0.14810505362112691, -0.06576904117079967,
      0.014026606298625958)


def _bce(xv, tf):
    ax = jnp.abs(xv)
    e = jnp.exp(-ax)
    q = jnp.float32(_Q[6]) * e + jnp.float32(_Q[5])
    for c in _Q[4::-1]:
        q = q * e + jnp.float32(c)
    return jnp.maximum(xv, jnp.float32(0.0)) + e * q - tf * xv


def _sc_reduce_body(x_hbm, t_hbm, out_hbm, xb, tb, accb):
    wid = lax.axis_index("s") * 2 + lax.axis_index("c")
    b = wid // _WPB
    r0 = (wid % _WPB) * _RPW

    def chunk_body(c, carry):
        pltpu.sync_copy(x_hbm.at[2 * b + 1, pl.ds(r0 + c * _CHR, _CHR), :], xb)
        pltpu.sync_copy(t_hbm.at[b, pl.ds(r0 + c * _CHR, _CHR), :], tb)

        def row_body(r, acc):
            def vec_body(j, acc2):
                s_all, s_pos, c_pos = acc2
                xv = xb[r, pl.ds(j * _L, _L)]
                tf = tb[r, pl.ds(j * _L, _L)].astype(jnp.float32)
                bce = _bce(xv, tf)
                pb = tf * bce
                return (s_all + bce, s_pos + pb, c_pos + tf)

            return lax.fori_loop(0, _W // _L, vec_body, acc)

        return lax.fori_loop(0, _CHR, row_body, carry)

    z = jnp.zeros((_L,), jnp.float32)
    s_all, s_pos, c_pos = lax.fori_loop(0, _NCH, chunk_body, (z, z, z))
    accb[pl.ds(0, _L)] = s_all
    accb[pl.ds(_L, _L)] = s_pos
    accb[pl.ds(2 * _L, _L)] = c_pos
    pltpu.sync_copy(accb, out_hbm.at[pl.ds(wid * 48, 48)])


@functools.lru_cache(maxsize=None)
def _sc_reduce():
    return pl.kernel(
        _sc_reduce_body,
        out_type=jax.ShapeDtypeStruct((_NW * 48,), jnp.float32),
        mesh=plsc.VectorSubcoreMesh(core_axis_name="c", subcore_axis_name="s"),
        scratch_types=[
            pltpu.VMEM((_CHR, _W), jnp.float32),
            pltpu.VMEM((_CHR, _W), jnp.int32),
            pltpu.VMEM((48,), jnp.float32),
        ],
        compiler_params=pltpu.CompilerParams(use_tc_tiling_on_sc=True),
    )


def _topk_body(fk_ref, x_hbm, t_hbm, out_ref, xb, tb, keys, sem):
    out_ref[0] = 0.0

    @pl.when(fk_ref[0] == 1)
    def _():
        def plane(b, carry):
            cpx = pltpu.make_async_copy(x_hbm.at[2 * b + 1], xb, sem)
            cpx.start()
            cpx.wait()
            cpt = pltpu.make_async_copy(t_hbm.at[b], tb, sem)
            cpt.start()
            cpt.wait()
            xv = xb[...]
            tv = tb[...]
            tf = tv.astype(jnp.float32)
            bce = (jnp.maximum(xv, 0.0) + jnp.log1p(jnp.exp(-jnp.abs(xv)))
                   - tf * xv)
            keys[pl.ds(b * _H, _H), :] = jnp.where(
                tv > 0, jnp.int32(0), pltpu.bitcast(bce, jnp.int32))
            return carry

        lax.fori_loop(0, _B, plane, 0)
        k = fk_ref[1]

        def bit_body(i, prefix):
            cand = prefix | jnp.left_shift(jnp.int32(1), 30 - i)
            cnt = jnp.sum((keys[...] >= cand).astype(jnp.int32))
            return jnp.where(cnt >= k, cand, prefix)

        prefix = lax.fori_loop(0, 31, bit_body, jnp.int32(0))
        kv = keys[...]
        vals = pltpu.bitcast(kv, jnp.float32)
        gt = kv > prefix
        cnt_gt = jnp.sum(gt.astype(jnp.int32))
        sum_gt = jnp.sum(jnp.where(gt, vals, 0.0))
        thr = jnp.max(jnp.where(kv == prefix, vals, 0.0))
        out_ref[0] = sum_gt + (k - cnt_gt).astype(jnp.float32) * thr


_topk_call = pl.pallas_call(
    _topk_body,
    in_specs=[
        pl.BlockSpec(memory_space=pltpu.SMEM),
        pl.BlockSpec(memory_space=pltpu.MemorySpace.HBM),
        pl.BlockSpec(memory_space=pltpu.MemorySpace.HBM),
    ],
    out_specs=pl.BlockSpec(memory_space=pltpu.SMEM),
    out_shape=jax.ShapeDtypeStruct((1,), jnp.float32),
    scratch_shapes=[
        pltpu.VMEM((_H, _W), jnp.float32),
        pltpu.VMEM((_H, _W), jnp.int32),
        pltpu.VMEM((_B * _H, _W), jnp.int32),
        pltpu.SemaphoreType.DMA,
    ],
)


def kernel(logits, target):
    x3 = logits.reshape(_B * _C, _H, _W)
    parts = _sc_reduce()(x3, target).reshape(_NW, 3, _L)
    s_all = jnp.sum(parts[:, 0, :])
    s_pos = jnp.sum(parts[:, 1, :])
    n_pos_f = jnp.sum(parts[:, 2, :])

    num_pos = n_pos_f.astype(jnp.int32)
    num_neg = jnp.int32(_N) - num_pos
    k_pos = jnp.minimum(num_neg, 20 * num_pos)
    k_empty = jnp.maximum(
        1, (num_neg.astype(jnp.float32) * jnp.float32(0.01)).astype(jnp.int32))
    k = jnp.where(num_pos > 0, k_pos, k_empty)

    run_topk = (k != num_neg).astype(jnp.int32)
    fk = jnp.stack([run_topk, k])
    topk_sum = jnp.where(
        run_topk == 1, _topk_call(fk, x3, target)[0], s_all - s_pos)
    neg_keep = jnp.where(
        num_neg > 0, topk_sum / jnp.maximum(k, 1).astype(jnp.float32), 0.0)
    pos_keep = jnp.where(
        num_pos > 0, s_pos / jnp.maximum(n_pos_f, 1.0), 0.0)
    return pos_keep + neg_keep
